# Initial kernel scaffold; baseline (speedup 1.0000x reference)
import functools

import jax
import jax.numpy as jnp
from jax import lax
from jax.experimental import pallas as pl
from jax.experimental.pallas import tpu as pltpu
from jax.experimental.pallas import tpu_sc as plsc

HID = 256
N = 10000
E = 160000
B = 256
NSUB = 16
K = 128
CHUNKS = 79
EPAD = NSUB * CHUNKS * K
NPAD = 10016
ROWS_PER_SUB = NPAD // NSUB
DUMMY_DST = N
R = 500
NB = N // R
_PREC = lax.Precision.HIGHEST


@functools.lru_cache(maxsize=None)
def _make_agg(ncols):
    mesh = plsc.VectorSubcoreMesh(core_axis_name="c", subcore_axis_name="s")
    out_type = (jax.ShapeDtypeStruct((N, ncols), jnp.float32),
                jax.ShapeDtypeStruct((N, ncols), jnp.float32))
    zrows = ROWS_PER_SUB // 2

    @functools.partial(
        pl.kernel, mesh=mesh, out_type=out_type,
        scratch_types=[
            pltpu.VMEM_SHARED((NPAD, ncols), jnp.float32),
            pltpu.VMEM((zrows, ncols), jnp.float32),
            pltpu.VMEM((K, ncols), jnp.float32),
            pltpu.VMEM((K, ncols), jnp.float32),
            pltpu.VMEM((K,), jnp.int32),
            pltpu.VMEM((K,), jnp.int32),
            pltpu.VMEM((K,), jnp.int32),
            pltpu.VMEM((K,), jnp.int32),
            pltpu.SemaphoreType.DMA,
            pltpu.SemaphoreType.DMA,
        ],
    )
    def agg(x0_hbm, x1_hbm, src_hbm, dst_hbm, a0_hbm, a1_hbm,
            accum, zbuf, rowsA, rowsB, srcA, srcB, dstA, dstB, gsemA, gsemB):
        c = lax.axis_index("c")
        s = lax.axis_index("s")
        ebase = s * (CHUNKS * K)
        r0 = s * ROWS_PER_SUB

        zc = ncols // 16

        @pl.loop(0, zrows)
        def _(i):
            @pl.loop(0, zc)
            def _(j):
                zbuf[i, pl.ds(j * 16, 16)] = jnp.zeros((16,), jnp.float32)

        pltpu.sync_copy(zbuf, accum.at[pl.ds(r0, zrows)])
        pltpu.sync_copy(zbuf, accum.at[pl.ds(r0 + zrows, zrows)])
        plsc.subcore_barrier()

        def run(x_hbm, a_hbm):
            pltpu.sync_copy(src_hbm.at[pl.ds(ebase, K)], srcA)
            pltpu.sync_copy(dst_hbm.at[pl.ds(ebase, K)], dstA)
            pltpu.async_copy(x_hbm.at[srcA], rowsA, gsemA)

            @pl.loop(0, CHUNKS - 1)
            def _(j):
                even = j % 2 == 0
                off = ebase + (j + 1) * K

                @pl.when(even)
                def _():
                    pltpu.sync_copy(src_hbm.at[pl.ds(off, K)], srcB)
                    pltpu.sync_copy(dst_hbm.at[pl.ds(off, K)], dstB)
                    pltpu.async_copy(x_hbm.at[srcB], rowsB, gsemB)
                    pltpu.make_async_copy(x_hbm.at[srcA], rowsA, gsemA).wait()
                    pltpu.sync_copy(rowsA, accum.at[dstA], add=True)

                @pl.when(jnp.logical_not(even))
                def _():
                    pltpu.sync_copy(src_hbm.at[pl.ds(off, K)], srcA)
                    pltpu.sync_copy(dst_hbm.at[pl.ds(off, K)], dstA)
                    pltpu.async_copy(x_hbm.at[srcA], rowsA, gsemA)
                    pltpu.make_async_copy(x_hbm.at[srcB], rowsB, gsemB).wait()
                    pltpu.sync_copy(rowsB, accum.at[dstB], add=True)

            if (CHUNKS - 1) % 2 == 1:
                pltpu.make_async_copy(x_hbm.at[srcA], rowsA, gsemA).wait()
                pltpu.sync_copy(rowsA, accum.at[dstA], add=True)
            else:
                pltpu.make_async_copy(x_hbm.at[srcB], rowsB, gsemB).wait()
                pltpu.sync_copy(rowsB, accum.at[dstB], add=True)

            plsc.subcore_barrier()
            nlast = N - (NSUB - 1) * ROWS_PER_SUB

            @pl.when(s < NSUB - 1)
            def _():
                pltpu.sync_copy(accum.at[pl.ds(r0, ROWS_PER_SUB)],
                                a_hbm.at[pl.ds(r0, ROWS_PER_SUB)])

            @pl.when(s == NSUB - 1)
            def _():
                pltpu.sync_copy(accum.at[pl.ds(r0, nlast)],
                                a_hbm.at[pl.ds(r0, nlast)])

        @pl.when(c == 0)
        def _():
            run(x0_hbm, a0_hbm)

        @pl.when(c == 1)
        def _():
            run(x1_hbm, a1_hbm)

    return agg


def _mm(a, b):
    return lax.dot_general(a, b, (((1,), (0,)), ((), ())), precision=_PREC)


def _mlp_phase1(x0, x1, a0, a1, W1, b1, W2, b2):
    cin = x0.shape[1]

    def kern(x0r, x1r, a0r, a1r, W1r, b1r, W2r, b2r, h2r, statr):
        i = pl.program_id(0)
        h = jnp.concatenate([x0r[...] + a0r[...], x1r[...] + a1r[...]], axis=1)
        z = jnp.maximum(_mm(h, W1r[...]) + b1r[...], 0.0)
        z = jnp.maximum(_mm(z, W2r[...]) + b2r[...], 0.0)
        h2r[...] = z

        @pl.when(i == 0)
        def _():
            statr[...] = jnp.zeros_like(statr)

        statr[0:1, :] += jnp.sum(z, axis=0, keepdims=True)
        statr[1:2, :] += jnp.sum(z * z, axis=0, keepdims=True)

    return pl.pallas_call(
        kern,
        grid=(NB,),
        in_specs=[
            pl.BlockSpec((R, cin), lambda i: (i, 0)),
            pl.BlockSpec((R, cin), lambda i: (i, 0)),
            pl.BlockSpec((R, cin), lambda i: (i, 0)),
            pl.BlockSpec((R, cin), lambda i: (i, 0)),
            pl.BlockSpec((2 * cin, HID), lambda i: (0, 0)),
            pl.BlockSpec((1, HID), lambda i: (0, 0)),
            pl.BlockSpec((HID, HID), lambda i: (0, 0)),
            pl.BlockSpec((1, HID), lambda i: (0, 0)),
        ],
        out_specs=[
            pl.BlockSpec((R, HID), lambda i: (i, 0)),
            pl.BlockSpec((8, HID), lambda i: (0, 0)),
        ],
        out_shape=[
            jax.ShapeDtypeStruct((N, HID), jnp.float32),
            jax.ShapeDtypeStruct((8, HID), jnp.float32),
        ],
    )(x0, x1, a0, a1, W1, b1, W2, b2)


def _mlp_phase2(h2, stats, gamma, beta):
    def kern(h2r, statr, gr, br, y0r, y1r):
        mu = statr[0:1, :] * (1.0 / N)
        var = statr[1:2, :] * (1.0 / N) - mu * mu
        inv = gr[...] * lax.rsqrt(var + 1e-5)
        yv = jnp.maximum((h2r[...] - mu) * inv + br[...], 0.0)
        y0r[...] = yv[:, :128]
        y1r[...] = yv[:, 128:]

    return pl.pallas_call(
        kern,
        grid=(NB,),
        in_specs=[
            pl.BlockSpec((R, HID), lambda i: (i, 0)),
            pl.BlockSpec((8, HID), lambda i: (0, 0)),
            pl.BlockSpec((1, HID), lambda i: (0, 0)),
            pl.BlockSpec((1, HID), lambda i: (0, 0)),
        ],
        out_specs=[
            pl.BlockSpec((R, 128), lambda i: (i, 0)),
            pl.BlockSpec((R, 128), lambda i: (i, 0)),
        ],
        out_shape=[
            jax.ShapeDtypeStruct((N, 128), jnp.float32),
            jax.ShapeDtypeStruct((N, 128), jnp.float32),
        ],
    )(h2, stats, gamma, beta)


def _pool(y0, y1, seg_row):
    def kern(y0r, y1r, segr, sumr, cntr):
        i = pl.program_id(0)

        @pl.when(i == 0)
        def _():
            sumr[...] = jnp.zeros_like(sumr)
            cntr[...] = jnp.zeros_like(cntr)

        seg = segr[...]
        gid = lax.broadcasted_iota(jnp.int32, (B, R), 0)
        oh = (gid == seg).astype(jnp.float32)
        yv = jnp.concatenate([y0r[...], y1r[...]], axis=1)
        sumr[...] += _mm(oh, yv)
        cntr[:, 0:1] += jnp.sum(oh, axis=1, keepdims=True)

    return pl.pallas_call(
        kern,
        grid=(NB,),
        in_specs=[
            pl.BlockSpec((R, 128), lambda i: (i, 0)),
            pl.BlockSpec((R, 128), lambda i: (i, 0)),
            pl.BlockSpec((1, R), lambda i: (0, i)),
        ],
        out_specs=[
            pl.BlockSpec((B, HID), lambda i: (0, 0)),
            pl.BlockSpec((B, 128), lambda i: (0, 0)),
        ],
        out_shape=[
            jax.ShapeDtypeStruct((B, HID), jnp.float32),
            jax.ShapeDtypeStruct((B, 128), jnp.float32),
        ],
    )(y0, y1, seg_row)


def _tail(lsum, lcnt, psum, pcnt, esm, wts):
    def kern(lsr, lcr, psr, pcr, esmr,
             Wvlr, bvlr, Wolr, bolr, Wvpr, bvpr, Wopr, bopr,
             eW1r, eb1r, eW2r, eb2r, fW1ar, fW1br, fW1cr, fb1r,
             fW2r, fb2r, pW1r, pb1r, pW2r, pb2r, outr):
        lmean = lsr[...] / jnp.maximum(lcr[:, 0:1], 1.0)
        pmean = psr[...] / jnp.maximum(pcr[:, 0:1], 1.0)
        lig_feat = _mm(_mm(pmean, Wvlr[...]) + bvlr[...], Wolr[...]) + bolr[...]
        prot_feat = _mm(_mm(lmean, Wvpr[...]) + bvpr[...], Wopr[...]) + bopr[...]
        e1 = jnp.maximum(_mm(esmr[...], eW1r[...]) + eb1r[...], 0.0)
        e2 = jnp.maximum(_mm(e1, eW2r[...]) + eb2r[...], 0.0)
        f1 = jnp.maximum(_mm(lig_feat, fW1ar[...]) + _mm(prot_feat, fW1br[...])
                         + _mm(e2, fW1cr[...]) + fb1r[...], 0.0)
        f2 = jnp.maximum(_mm(f1, fW2r[...]) + fb2r[...], 0.0)
        hh = jnp.maximum(_mm(f2, pW1r[...]) + pb1r[...], 0.0)
        outr[...] = _mm(hh, pW2r[...]) + pb2r[...]

    return pl.pallas_call(
        kern,
        out_shape=jax.ShapeDtypeStruct((B, 128), jnp.float32),
    )(lsum, lcnt, psum, pcnt, esm, *wts)


def _row(v):
    return v.reshape(1, -1)


def kernel(ligand_x, ligand_edge_index, ligand_batch, protein_x,
           protein_edge_index, protein_batch, esm_embedding, y, params):
    agg16 = _make_agg(16)
    agg128 = _make_agg(128)

    def prep_edges(ei):
        pad = EPAD - E
        src = jnp.concatenate([ei[0], jnp.zeros((pad,), jnp.int32)])
        dst = jnp.concatenate([ei[1], jnp.full((pad,), DUMMY_DST, jnp.int32)])
        return src, dst

    def gin_stack(x, ei, layers):
        nf = x.shape[1]
        xp = jnp.pad(x, ((0, 0), (0, 32 - nf)))
        x0, x1 = xp[:, :16], xp[:, 16:]
        src, dst = prep_edges(ei)
        for li, lp in enumerate(layers):
            if li == 0:
                a0, a1 = agg16(x0, x1, src, dst)
                W1 = jnp.pad(lp["W1"], ((0, 32 - nf), (0, 0)))
            else:
                a0, a1 = agg128(x0, x1, src, dst)
                W1 = lp["W1"]
            h2, stats = _mlp_phase1(x0, x1, a0, a1, W1, _row(lp["b1"]),
                                    lp["W2"], _row(lp["b2"]))
            x0, x1 = _mlp_phase2(h2, stats, _row(lp["gamma"]), _row(lp["beta"]))
        return x0, x1

    p = params
    l0, l1 = gin_stack(ligand_x, ligand_edge_index, p["lig_gin"])
    p0, p1 = gin_stack(protein_x, protein_edge_index, p["prot_gin"])

    lsum, lcnt = _pool(l0, l1, _row(ligand_batch))
    psum, pcnt = _pool(p0, p1, _row(protein_batch))

    al, ap, pe, pf, pp = (p["attn_lig"], p["attn_prot"], p["esm"],
                          p["fusion"], p["pred"])
    fW1 = pf["W1"]
    pW2 = jnp.pad(pp["W2"], ((0, 0), (0, 127)))
    pb2 = jnp.pad(_row(pp["b2"]), ((0, 0), (0, 127)))
    wts = (al["Wv"], _row(al["bv"]), al["Wo"], _row(al["bo"]),
           ap["Wv"], _row(ap["bv"]), ap["Wo"], _row(ap["bo"]),
           pe["W1"], _row(pe["b1"]), pe["W2"], _row(pe["b2"]),
           fW1[:HID], fW1[HID:2 * HID], fW1[2 * HID:], _row(pf["b1"]),
           pf["W2"], _row(pf["b2"]), pp["W1"], _row(pp["b1"]), pW2, pb2)
    out = _tail(lsum, lcnt, psum, pcnt, esm_embedding, wts)
    return out[:, :1]


# speedup vs baseline: 2.0209x; 2.0209x over previous
import functools

import jax
import jax.numpy as jnp
from jax import lax
from jax.experimental import pallas as pl
from jax.experimental.pallas import tpu as pltpu
from jax.experimental.pallas import tpu_sc as plsc

HID = 256
N = 10000
E = 160000
B = 256
NSUB = 16
K = 128
EPAD = E + 256
NPAD = 10112
ROWS_PER_SUB = NPAD // NSUB
DUMMY_DST = N
DUMMY2 = NPAD - 8
R = 400
NB = N // R
_DEF = lax.Precision.DEFAULT
_HI = lax.Precision.HIGHEST


@functools.lru_cache(maxsize=None)
def _make_agg(ncols):
    mesh = plsc.VectorSubcoreMesh(core_axis_name="c", subcore_axis_name="s")
    out_type = (jax.ShapeDtypeStruct((N, ncols), jnp.float32),
                jax.ShapeDtypeStruct((N, ncols), jnp.float32))

    @functools.partial(
        pl.kernel, mesh=mesh, out_type=out_type,
        scratch_types=[
            pltpu.VMEM_SHARED((NPAD, ncols), jnp.float32),
            pltpu.VMEM((K, ncols), jnp.float32),
            pltpu.VMEM((K,), jnp.int32),
            pltpu.VMEM((K,), jnp.int32),
            pltpu.VMEM((16,), jnp.int32),
            pltpu.SemaphoreType.DMA,
        ],
    )
    def agg(x0_hbm, x1_hbm, src_hbm, dst_hbm, bounds_hbm, a0_hbm, a1_hbm,
            accum, rowsA, srcA, dstA, bv, gsemA):
        c = lax.axis_index("c")
        s = lax.axis_index("s")
        r0 = s * ROWS_PER_SUB

        zc = ncols // 16

        @pl.loop(0, K)
        def _(i):
            @pl.loop(0, zc)
            def _(j):
                rowsA[i, pl.ds(j * 16, 16)] = jnp.zeros((16,), jnp.float32)

        nz = ROWS_PER_SUB // K

        @pl.loop(0, nz)
        def _(i):
            pltpu.sync_copy(rowsA, accum.at[pl.ds(r0 + i * K, K)])

        ztail = ROWS_PER_SUB - nz * K
        if ztail:
            pltpu.sync_copy(rowsA.at[pl.ds(0, ztail)],
                            accum.at[pl.ds(r0 + nz * K, ztail)])
        plsc.subcore_barrier()

        pltpu.sync_copy(bounds_hbm.at[s], bv)
        bvv = bv[...]
        start = bvv[0]
        end = bvv[1]
        base0 = (start // 8) * 8
        nch = (end - base0 + (K - 1)) // K

        def run(x_hbm, a_hbm):
            @pl.loop(0, nch)
            def _(j):
                off = base0 + j * K
                pltpu.sync_copy(src_hbm.at[pl.ds(off, K)], srcA)
                pltpu.sync_copy(dst_hbm.at[pl.ds(off, K)], dstA)
                pltpu.async_copy(x_hbm.at[srcA], rowsA, gsemA).wait()
                for k in range(K // 16):
                    pos = off + k * 16 + lax.iota(jnp.int32, 16)
                    valid = (pos >= start) & (pos < end)
                    d = dstA[pl.ds(k * 16, 16)]
                    dstA[pl.ds(k * 16, 16)] = jnp.where(
                        valid, d, jnp.full((16,), DUMMY2, jnp.int32))
                pltpu.sync_copy(rowsA, accum.at[dstA], add=True)

            plsc.subcore_barrier()
            nlast = N - (NSUB - 1) * ROWS_PER_SUB

            @pl.when(s < NSUB - 1)
            def _():
                pltpu.sync_copy(accum.at[pl.ds(r0, ROWS_PER_SUB)],
                                a_hbm.at[pl.ds(r0, ROWS_PER_SUB)])

            @pl.when(s == NSUB - 1)
            def _():
                pltpu.sync_copy(accum.at[pl.ds(r0, nlast)],
                                a_hbm.at[pl.ds(r0, nlast)])

        @pl.when(c == 0)
        def _():
            run(x0_hbm, a0_hbm)

        @pl.when(c == 1)
        def _():
            run(x1_hbm, a1_hbm)

    return agg


def _mm(a, b, prec=_DEF):
    return lax.dot_general(a, b, (((1,), (0,)), ((), ())), precision=prec)


def _stats_update(i, z, h2r, statr):
    h2r[...] = z

    @pl.when(i == 0)
    def _():
        statr[...] = jnp.zeros_like(statr)

    statr[0:1, :] += jnp.sum(z, axis=0, keepdims=True)


_MLP_OUT_SPECS = [
    pl.BlockSpec((R, HID), lambda i: (i, 0)),
    pl.BlockSpec((8, HID), lambda i: (0, 0)),
]
_MLP_OUT_SHAPE = [
    jax.ShapeDtypeStruct((N, HID), jnp.float32),
    jax.ShapeDtypeStruct((8, HID), jnp.float32),
]


def _mlp_first(xp, a0, W1p, b1, W2, b2):
    cin = xp.shape[1]

    def kern(xr, a0r, W1r, b1r, W2r, b2r, h2r, statr):
        i = pl.program_id(0)
        h = xr[...] + a0r[...][:, :cin]
        z = jnp.maximum(_mm(h, W1r[...], _HI) + b1r[...], 0.0)
        z = jnp.maximum(_mm(z, W2r[...], _HI) + b2r[...], 0.0)
        _stats_update(i, z, h2r, statr)

    return pl.pallas_call(
        kern,
        grid=(NB,),
        in_specs=[
            pl.BlockSpec((R, cin), lambda i: (i, 0)),
            pl.BlockSpec((R, 128), lambda i: (i, 0)),
            pl.BlockSpec((cin, HID), lambda i: (0, 0)),
            pl.BlockSpec((1, HID), lambda i: (0, 0)),
            pl.BlockSpec((HID, HID), lambda i: (0, 0)),
            pl.BlockSpec((1, HID), lambda i: (0, 0)),
        ],
        out_specs=_MLP_OUT_SPECS,
        out_shape=_MLP_OUT_SHAPE,
    )(xp, a0, W1p, b1, W2, b2)


def _mlp_phase1(x0, x1, a0, a1, W1, b1, W2, b2, w1_prec=_DEF, w2_prec=_DEF):
    def kern(x0r, x1r, a0r, a1r, W1r, b1r, W2r, b2r, h2r, statr):
        i = pl.program_id(0)
        h = jnp.concatenate([x0r[...] + a0r[...], x1r[...] + a1r[...]], axis=1)
        z = jnp.maximum(_mm(h, W1r[...], w1_prec) + b1r[...], 0.0)
        z = jnp.maximum(_mm(z, W2r[...], w2_prec) + b2r[...], 0.0)
        _stats_update(i, z, h2r, statr)

    return pl.pallas_call(
        kern,
        grid=(NB,),
        in_specs=[
            pl.BlockSpec((R, 128), lambda i: (i, 0)),
            pl.BlockSpec((R, 128), lambda i: (i, 0)),
            pl.BlockSpec((R, 128), lambda i: (i, 0)),
            pl.BlockSpec((R, 128), lambda i: (i, 0)),
            pl.BlockSpec((HID, HID), lambda i: (0, 0)),
            pl.BlockSpec((1, HID), lambda i: (0, 0)),
            pl.BlockSpec((HID, HID), lambda i: (0, 0)),
            pl.BlockSpec((1, HID), lambda i: (0, 0)),
        ],
        out_specs=_MLP_OUT_SPECS,
        out_shape=_MLP_OUT_SHAPE,
    )(x0, x1, a0, a1, W1, b1, W2, b2)


def _bn_var(h2, stats):
    def kern(h2r, statr, vr):
        i = pl.program_id(0)

        @pl.when(i == 0)
        def _():
            vr[...] = jnp.zeros_like(vr)

        dz = h2r[...] - statr[0:1, :] * (1.0 / N)
        vr[0:1, :] += jnp.sum(dz * dz, axis=0, keepdims=True)

    return pl.pallas_call(
        kern,
        grid=(NB,),
        in_specs=[
            pl.BlockSpec((R, HID), lambda i: (i, 0)),
            pl.BlockSpec((8, HID), lambda i: (0, 0)),
        ],
        out_specs=pl.BlockSpec((8, HID), lambda i: (0, 0)),
        out_shape=jax.ShapeDtypeStruct((8, HID), jnp.float32),
    )(h2, stats)


def _mlp_phase2(h2, stats, varstats, gamma, beta):
    def kern(h2r, statr, vr, gr, br, y0r, y1r):
        mu = statr[0:1, :] * (1.0 / N)
        var = vr[0:1, :] * (1.0 / N)
        yv = jnp.maximum(gr[...] * (h2r[...] - mu) / jnp.sqrt(var + 1e-5)
                         + br[...], 0.0)
        y0r[...] = yv[:, :128]
        y1r[...] = yv[:, 128:]

    return pl.pallas_call(
        kern,
        grid=(NB,),
        in_specs=[
            pl.BlockSpec((R, HID), lambda i: (i, 0)),
            pl.BlockSpec((8, HID), lambda i: (0, 0)),
            pl.BlockSpec((8, HID), lambda i: (0, 0)),
            pl.BlockSpec((1, HID), lambda i: (0, 0)),
            pl.BlockSpec((1, HID), lambda i: (0, 0)),
        ],
        out_specs=[
            pl.BlockSpec((R, 128), lambda i: (i, 0)),
            pl.BlockSpec((R, 128), lambda i: (i, 0)),
        ],
        out_shape=[
            jax.ShapeDtypeStruct((N, 128), jnp.float32),
            jax.ShapeDtypeStruct((N, 128), jnp.float32),
        ],
    )(h2, stats, varstats, gamma, beta)


def _pool(y0, y1, seg_row):
    def kern(y0r, y1r, segr, sumr, cntr):
        i = pl.program_id(0)

        @pl.when(i == 0)
        def _():
            sumr[...] = jnp.zeros_like(sumr)
            cntr[...] = jnp.zeros_like(cntr)

        seg = segr[0]
        gid = lax.broadcasted_iota(jnp.int32, (B, R), 0)
        oh = (gid == seg).astype(jnp.float32)
        yv = jnp.concatenate([y0r[...], y1r[...]], axis=1)
        sumr[...] += _mm(oh, yv, _HI)
        cntr[:, 0:1] += jnp.sum(oh, axis=1, keepdims=True)

    return pl.pallas_call(
        kern,
        grid=(NB,),
        in_specs=[
            pl.BlockSpec((R, 128), lambda i: (i, 0)),
            pl.BlockSpec((R, 128), lambda i: (i, 0)),
            pl.BlockSpec((1, 1, R), lambda i: (i, 0, 0)),
        ],
        out_specs=[
            pl.BlockSpec((B, HID), lambda i: (0, 0)),
            pl.BlockSpec((B, 128), lambda i: (0, 0)),
        ],
        out_shape=[
            jax.ShapeDtypeStruct((B, HID), jnp.float32),
            jax.ShapeDtypeStruct((B, 128), jnp.float32),
        ],
    )(y0, y1, seg_row)


def _tail(lsum, lcnt, psum, pcnt, esm, wts):
    def kern(lsr, lcr, psr, pcr, esmr,
             Wvlr, bvlr, Wolr, bolr, Wvpr, bvpr, Wopr, bopr,
             eW1r, eb1r, eW2r, eb2r, fW1ar, fW1br, fW1cr, fb1r,
             fW2r, fb2r, pW1r, pb1r, pW2r, pb2r, outr):
        lmean = lsr[...] / jnp.maximum(lcr[:, 0:1], 1.0)
        pmean = psr[...] / jnp.maximum(pcr[:, 0:1], 1.0)
        lig_feat = _mm(_mm(pmean, Wvlr[...]) + bvlr[...], Wolr[...]) + bolr[...]
        prot_feat = _mm(_mm(lmean, Wvpr[...]) + bvpr[...], Wopr[...]) + bopr[...]
        e1 = jnp.maximum(_mm(esmr[...], eW1r[...]) + eb1r[...], 0.0)
        e2 = jnp.maximum(_mm(e1, eW2r[...]) + eb2r[...], 0.0)
        f1 = jnp.maximum(_mm(lig_feat, fW1ar[...]) + _mm(prot_feat, fW1br[...])
                         + _mm(e2, fW1cr[...]) + fb1r[...], 0.0)
        f2 = jnp.maximum(_mm(f1, fW2r[...]) + fb2r[...], 0.0)
        hh = jnp.maximum(_mm(f2, pW1r[...]) + pb1r[...], 0.0)
        outr[...] = _mm(hh, pW2r[...]) + pb2r[...]

    return pl.pallas_call(
        kern,
        out_shape=jax.ShapeDtypeStruct((B, 128), jnp.float32),
    )(lsum, lcnt, psum, pcnt, esm, *wts)


def _row(v):
    return v.reshape(1, -1)


def kernel(ligand_x, ligand_edge_index, ligand_batch, protein_x,
           protein_edge_index, protein_batch, esm_embedding, y, params):
    agg128 = _make_agg(128)
    def prep_edges(ei):
        perm = jnp.argsort(ei[1], stable=True)
        src = jnp.concatenate([ei[0][perm], jnp.zeros((EPAD - E,), jnp.int32)])
        dst = jnp.concatenate([ei[1][perm],
                               jnp.full((EPAD - E,), DUMMY_DST, jnp.int32)])
        marks = jnp.arange(NSUB, dtype=jnp.int32) * ROWS_PER_SUB
        lefts = jnp.searchsorted(dst, marks).astype(jnp.int32)
        rights = jnp.searchsorted(dst, marks + ROWS_PER_SUB).astype(jnp.int32)
        rights = jnp.minimum(rights, E + 128)
        bounds = (jnp.zeros((NSUB, 16), jnp.int32)
                  .at[:, 0].set(lefts).at[:, 1].set(rights))
        return src, dst, bounds

    def gin_stack(x, ei, layers):
        nf = x.shape[1]
        xp = jnp.pad(x, ((0, 0), (0, 32 - nf)))
        xp128 = jnp.pad(x, ((0, 0), (0, 128 - nf)))
        W1p = jnp.pad(layers[0]["W1"], ((0, 32 - nf), (0, 0)))
        src, dst, bounds = prep_edges(ei)
        ap0, _ = agg128(xp128, xp128, src, dst, bounds)
        lp = layers[0]
        h2, stats = _mlp_first(xp, ap0, W1p, _row(lp["b1"]),
                               lp["W2"], _row(lp["b2"]))
        x0, x1 = _mlp_phase2(h2, stats, _bn_var(h2, stats),
                             _row(lp["gamma"]), _row(lp["beta"]))
        for lp in layers[1:]:
            a0, a1 = agg128(x0, x1, src, dst, bounds)
            h2, stats = _mlp_phase1(x0, x1, a0, a1, lp["W1"], _row(lp["b1"]),
                                    lp["W2"], _row(lp["b2"]))
            x0, x1 = _mlp_phase2(h2, stats, _bn_var(h2, stats),
                                 _row(lp["gamma"]), _row(lp["beta"]))
        return x0, x1

    p = params
    l0, l1 = gin_stack(ligand_x, ligand_edge_index, p["lig_gin"])
    p0, p1 = gin_stack(protein_x, protein_edge_index, p["prot_gin"])

    lsum, lcnt = _pool(l0, l1, ligand_batch.reshape(NB, 1, R))
    psum, pcnt = _pool(p0, p1, protein_batch.reshape(NB, 1, R))

    al, ap, pe, pf, pp = (p["attn_lig"], p["attn_prot"], p["esm"],
                          p["fusion"], p["pred"])
    fW1 = pf["W1"]
    pW2 = jnp.pad(pp["W2"], ((0, 0), (0, 127)))
    pb2 = jnp.pad(_row(pp["b2"]), ((0, 0), (0, 127)))
    wts = (al["Wv"], _row(al["bv"]), al["Wo"], _row(al["bo"]),
           ap["Wv"], _row(ap["bv"]), ap["Wo"], _row(ap["bo"]),
           pe["W1"], _row(pe["b1"]), pe["W2"], _row(pe["b2"]),
           fW1[:HID], fW1[HID:2 * HID], fW1[2 * HID:], _row(pf["b1"]),
           pf["W2"], _row(pf["b2"]), pp["W1"], _row(pp["b1"]), pW2, pb2)
    out = _tail(lsum, lcnt, psum, pcnt, esm_embedding, wts)
    return out[:, :1]
